# Initial kernel scaffold; baseline (speedup 1.0000x reference)
import jax
import jax.numpy as jnp
from jax.experimental import pallas as pl


def kernel(positions, hidden_states, kv_cache, residual, ln1_w, qkv_w, o_w, ln2_w, router_w, w1, w3, w2):
    raise NotImplementedError("write your pallas kernel here")


# speedup vs baseline: 1.0675x; 1.0675x over previous
import functools

import jax
import jax.numpy as jnp
from jax import lax
from jax.experimental import pallas as pl
from jax.experimental.pallas import tpu as pltpu

T = 2048
D = 1024
NH = 16
NKV = 8
HD = 64
E = 8
TOPK = 2
F = 512
EPS = 1e-06

_F32 = jnp.float32


_BF16 = jnp.bfloat16


def _dot_t(a, b):
    return lax.dot_general(a.astype(_BF16), b.astype(_BF16),
                           (((1,), (1,)), ((), ())),
                           preferred_element_type=_F32)


def _dot(a, b):
    return lax.dot_general(a.astype(_BF16), b.astype(_BF16),
                           (((1,), (0,)), ((), ())),
                           preferred_element_type=_F32)


def _qkv_body(h_ref, r_ref, w_ref, ln_ref, res_ref, qkv_ref):
    res = h_ref[...] + r_ref[...]
    res_ref[...] = res
    ms = jnp.mean(res * res, axis=1, keepdims=True)
    hn = res * lax.rsqrt(ms + EPS) * ln_ref[...]
    qkv_ref[...] = _dot_t(hn, w_ref[...])


def _fused_qkv(hidden, residual, qkv_w, ln1_w, bt=256):
    grid = (T // bt,)
    return pl.pallas_call(
        _qkv_body,
        grid=grid,
        in_specs=[
            pl.BlockSpec((bt, D), lambda i: (i, 0)),
            pl.BlockSpec((bt, D), lambda i: (i, 0)),
            pl.BlockSpec((NH * HD + 2 * NKV * HD, D), lambda i: (0, 0)),
            pl.BlockSpec((1, D), lambda i: (0, 0)),
        ],
        out_specs=[
            pl.BlockSpec((bt, D), lambda i: (i, 0)),
            pl.BlockSpec((bt, NH * HD + 2 * NKV * HD), lambda i: (i, 0)),
        ],
        out_shape=[
            jax.ShapeDtypeStruct((T, D), _F32),
            jax.ShapeDtypeStruct((T, NH * HD + 2 * NKV * HD), _F32),
        ],
    )(hidden, residual, qkv_w, ln1_w.reshape(1, D))


def _attn_body(q_ref, k_ref, v_ref, o_ref, *, qb):
    qi = pl.program_id(1)
    q = q_ref[0]
    k = k_ref[0]
    s = _dot_t(q, k) * (HD ** -0.5)
    rows = qi * qb + lax.broadcasted_iota(jnp.int32, s.shape, 0)
    cols = lax.broadcasted_iota(jnp.int32, s.shape, 1)
    s = jnp.where(cols <= rows, s, jnp.finfo(_F32).min)
    m = jnp.max(s, axis=1, keepdims=True)
    p = jnp.exp(s - m)
    p = p / jnp.sum(p, axis=1, keepdims=True)
    o_ref[0] = _dot(p, v_ref[0])


def _flash_attention(q, k, v, qb=512):
    nq = T // qb
    grid = (NH, nq)
    body = functools.partial(_attn_body, qb=qb)
    return pl.pallas_call(
        body,
        grid=grid,
        in_specs=[
            pl.BlockSpec((1, qb, HD), lambda h, qi: (h, qi, 0)),
            pl.BlockSpec((1, T, HD), lambda h, qi: (h // 2, 0, 0)),
            pl.BlockSpec((1, T, HD), lambda h, qi: (h // 2, 0, 0)),
        ],
        out_specs=pl.BlockSpec((1, qb, HD), lambda h, qi: (h, qi, 0)),
        out_shape=jax.ShapeDtypeStruct((NH, T, HD), _F32),
    )(q, k, v)


def _post_body(a_ref, res_ref, ow_ref, ln_ref, rw_ref,
               res2_ref, h2_ref, comb_ref):
    o = _dot_t(a_ref[...], ow_ref[...])
    res2 = o + res_ref[...]
    res2_ref[...] = res2
    ms = jnp.mean(res2 * res2, axis=1, keepdims=True)
    h2 = res2 * lax.rsqrt(ms + EPS) * ln_ref[...]
    h2_ref[...] = h2
    logits = _dot_t(h2, rw_ref[...])
    logits = logits - jnp.max(logits, axis=1, keepdims=True)
    ex = jnp.exp(logits)
    probs = ex / jnp.sum(ex, axis=1, keepdims=True)
    iota = lax.broadcasted_iota(jnp.int32, probs.shape, 1)
    m1 = jnp.max(probs, axis=1, keepdims=True)
    i1 = jnp.min(jnp.where(probs == m1, iota, E), axis=1, keepdims=True)
    oh1 = iota == i1
    probs2 = jnp.where(oh1, -1.0, probs)
    m2 = jnp.max(probs2, axis=1, keepdims=True)
    i2 = jnp.min(jnp.where(probs2 == m2, iota, E), axis=1, keepdims=True)
    oh2 = iota == i2
    comb_ref[...] = jnp.where(oh1 | oh2, probs, 0.0)


def _post_attn(attn, res, o_w, ln2_w, router_w, bt=256):
    grid = (T // bt,)
    return pl.pallas_call(
        _post_body,
        grid=grid,
        in_specs=[
            pl.BlockSpec((bt, NH * HD), lambda i: (i, 0)),
            pl.BlockSpec((bt, D), lambda i: (i, 0)),
            pl.BlockSpec((D, NH * HD), lambda i: (0, 0)),
            pl.BlockSpec((1, D), lambda i: (0, 0)),
            pl.BlockSpec((E, D), lambda i: (0, 0)),
        ],
        out_specs=[
            pl.BlockSpec((bt, D), lambda i: (i, 0)),
            pl.BlockSpec((bt, D), lambda i: (i, 0)),
            pl.BlockSpec((bt, E), lambda i: (i, 0)),
        ],
        out_shape=[
            jax.ShapeDtypeStruct((T, D), _F32),
            jax.ShapeDtypeStruct((T, D), _F32),
            jax.ShapeDtypeStruct((T, E), _F32),
        ],
    )(attn, res, o_w, ln2_w.reshape(1, D), router_w)


def _moe_body(x_ref, w1_ref, w3_ref, w2_ref, comb_ref, out_ref):
    e = pl.program_id(1)
    x = x_ref[...]
    g = _dot(x, w1_ref[0])
    u = _dot(x, w3_ref[0])
    act = g * jax.nn.sigmoid(g) * u
    y = _dot(act, w2_ref[0])
    comb = comb_ref[...]
    lane = lax.broadcasted_iota(jnp.int32, comb.shape, 1)
    w = jnp.sum(jnp.where(lane == e, comb, 0.0), axis=1, keepdims=True)

    @pl.when(e == 0)
    def _init():
        out_ref[...] = jnp.zeros_like(out_ref)

    out_ref[...] += w * y


def _moe(h2, combine, w1, w3, w2, bt=256):
    grid = (T // bt, E)
    return pl.pallas_call(
        _moe_body,
        grid=grid,
        in_specs=[
            pl.BlockSpec((bt, D), lambda t, e: (t, 0)),
            pl.BlockSpec((1, D, F), lambda t, e: (e, 0, 0)),
            pl.BlockSpec((1, D, F), lambda t, e: (e, 0, 0)),
            pl.BlockSpec((1, F, D), lambda t, e: (e, 0, 0)),
            pl.BlockSpec((bt, E), lambda t, e: (t, 0)),
        ],
        out_specs=pl.BlockSpec((bt, D), lambda t, e: (t, 0)),
        out_shape=jax.ShapeDtypeStruct((T, D), _F32),
    )(h2, w1, w3, w2, combine)


def kernel(positions, hidden_states, kv_cache, residual, ln1_w, qkv_w, o_w,
           ln2_w, router_w, w1, w3, w2):
    res, qkv = _fused_qkv(hidden_states, residual, qkv_w, ln1_w)
    q = qkv[:, :NH * HD].reshape(T, NH, HD).transpose(1, 0, 2)
    k = qkv[:, NH * HD:NH * HD + NKV * HD].reshape(T, NKV, HD).transpose(1, 0, 2)
    v = qkv[:, NH * HD + NKV * HD:].reshape(T, NKV, HD).transpose(1, 0, 2)
    attn3 = _flash_attention(q, k, v)
    attn = attn3.transpose(1, 0, 2).reshape(T, NH * HD)
    res2, h2, combine = _post_attn(attn, res, o_w, ln2_w, router_w)
    ff = _moe(h2, combine, w1, w3, w2)
    return (ff, res2)


# speedup vs baseline: 1.2959x; 1.2140x over previous
import functools

import jax
import jax.numpy as jnp
from jax import lax
from jax.experimental import pallas as pl
from jax.experimental.pallas import tpu as pltpu

T = 2048
D = 1024
NH = 16
NKV = 8
HD = 64
E = 8
TOPK = 2
F = 512
EPS = 1e-06

_F32 = jnp.float32


_BF16 = jnp.bfloat16


def _dot_t(a, b):
    return lax.dot_general(a.astype(_BF16), b.astype(_BF16),
                           (((1,), (1,)), ((), ())),
                           preferred_element_type=_F32)


def _dot(a, b):
    return lax.dot_general(a.astype(_BF16), b.astype(_BF16),
                           (((1,), (0,)), ((), ())),
                           preferred_element_type=_F32)


def _qkv_body(h_ref, r_ref, w_ref, ln_ref, res_ref, qkv_ref):
    res = h_ref[...] + r_ref[...]
    res_ref[...] = res
    ms = jnp.mean(res * res, axis=1, keepdims=True)
    hn = res * lax.rsqrt(ms + EPS) * ln_ref[...]
    qkv_ref[...] = _dot_t(hn, w_ref[...])


def _fused_qkv(hidden, residual, qkv_w, ln1_w, bt=256):
    grid = (T // bt,)
    return pl.pallas_call(
        _qkv_body,
        grid=grid,
        in_specs=[
            pl.BlockSpec((bt, D), lambda i: (i, 0)),
            pl.BlockSpec((bt, D), lambda i: (i, 0)),
            pl.BlockSpec((NH * HD + 2 * NKV * HD, D), lambda i: (0, 0)),
            pl.BlockSpec((1, D), lambda i: (0, 0)),
        ],
        out_specs=[
            pl.BlockSpec((bt, D), lambda i: (i, 0)),
            pl.BlockSpec((bt, NH * HD + 2 * NKV * HD), lambda i: (i, 0)),
        ],
        out_shape=[
            jax.ShapeDtypeStruct((T, D), _F32),
            jax.ShapeDtypeStruct((T, NH * HD + 2 * NKV * HD), _F32),
        ],
    )(hidden, residual, qkv_w, ln1_w.reshape(1, D))


def _attn_body(qkv_ref, k2_ref, v2_ref, o_ref, s_scr, m_scr, l_scr, a_scr,
               *, qb, kb, nk):
    g = pl.program_id(0)
    qi = pl.program_id(1)
    scale = HD ** -0.5
    fmin = jnp.finfo(_F32).min

    def head(hh, kh):
        q = qkv_ref[:, pl.ds(64 * hh, 64)]
        m_scr[...] = jnp.full_like(m_scr, fmin)
        l_scr[...] = jnp.zeros_like(l_scr)
        for kc in range(nk):
            @pl.when(kc <= qi)
            def _s():
                k = k2_ref[pl.ds(kc * kb, kb), pl.ds(64 * kh, 64)]
                s = _dot_t(q, k) * scale
                rows = qi * qb + lax.broadcasted_iota(jnp.int32, s.shape, 0)
                cols = kc * kb + lax.broadcasted_iota(jnp.int32, s.shape, 1)
                s = jnp.where(cols <= rows, s, fmin)
                s_scr[:, pl.ds(kc * kb, kb)] = s
                m_scr[:, pl.ds(kc, 1)] = jnp.max(s, axis=1, keepdims=True)

        m = jnp.max(m_scr[...], axis=1, keepdims=True)
        for kc in range(nk):
            @pl.when(kc <= qi)
            def _p():
                p = jnp.exp(s_scr[:, pl.ds(kc * kb, kb)] - m)
                s_scr[:, pl.ds(kc * kb, kb)] = p
                l_scr[:, pl.ds(kc, 1)] = jnp.sum(p, axis=1, keepdims=True)

        l = jnp.sum(l_scr[...], axis=1, keepdims=True)
        a_scr[...] = jnp.zeros_like(a_scr)
        for kc in range(nk):
            @pl.when(kc <= qi)
            def _pv():
                probs = s_scr[:, pl.ds(kc * kb, kb)] / l
                v = v2_ref[pl.ds(kc * kb, kb), pl.ds(64 * kh, 64)]
                a_scr[...] += _dot(probs, v)
        o_ref[:, pl.ds(64 * hh, 64)] = a_scr[...]

    @pl.when(g % 2 == 0)
    def _even():
        head(0, 0)
        head(1, 0)

    @pl.when(g % 2 == 1)
    def _odd():
        head(0, 1)
        head(1, 1)


def _attention(qkv, qb=512, kb=512):
    nq = T // qb
    nk = T // kb
    grid = (NH // 2, nq)
    body = functools.partial(_attn_body, qb=qb, kb=kb, nk=nk)
    return pl.pallas_call(
        body,
        grid=grid,
        in_specs=[
            pl.BlockSpec((qb, 128), lambda g, qi: (qi, g)),
            pl.BlockSpec((T, 128), lambda g, qi: (0, 8 + g // 2)),
            pl.BlockSpec((T, 128), lambda g, qi: (0, 12 + g // 2)),
        ],
        out_specs=pl.BlockSpec((qb, 128), lambda g, qi: (qi, g)),
        out_shape=jax.ShapeDtypeStruct((T, NH * HD), _F32),
        scratch_shapes=[
            pltpu.VMEM((qb, T), _F32),
            pltpu.VMEM((qb, nk), _F32),
            pltpu.VMEM((qb, nk), _F32),
            pltpu.VMEM((qb, HD), _F32),
        ],
    )(qkv, qkv, qkv)


def _post_body(a_ref, res_ref, ow_ref, ln_ref, rw_ref,
               res2_ref, h2_ref, comb_ref):
    o = _dot_t(a_ref[...], ow_ref[...])
    res2 = o + res_ref[...]
    res2_ref[...] = res2
    ms = jnp.mean(res2 * res2, axis=1, keepdims=True)
    h2 = res2 * lax.rsqrt(ms + EPS) * ln_ref[...]
    h2_ref[...] = h2
    logits = _dot_t(h2, rw_ref[...])
    logits = logits - jnp.max(logits, axis=1, keepdims=True)
    ex = jnp.exp(logits)
    probs = ex / jnp.sum(ex, axis=1, keepdims=True)
    iota = lax.broadcasted_iota(jnp.int32, probs.shape, 1)
    m1 = jnp.max(probs, axis=1, keepdims=True)
    i1 = jnp.min(jnp.where(probs == m1, iota, E), axis=1, keepdims=True)
    oh1 = iota == i1
    probs2 = jnp.where(oh1, -1.0, probs)
    m2 = jnp.max(probs2, axis=1, keepdims=True)
    i2 = jnp.min(jnp.where(probs2 == m2, iota, E), axis=1, keepdims=True)
    oh2 = iota == i2
    comb_ref[...] = jnp.where(oh1 | oh2, probs, 0.0)


def _post_attn(attn, res, o_w, ln2_w, router_w, bt=256):
    grid = (T // bt,)
    return pl.pallas_call(
        _post_body,
        grid=grid,
        in_specs=[
            pl.BlockSpec((bt, NH * HD), lambda i: (i, 0)),
            pl.BlockSpec((bt, D), lambda i: (i, 0)),
            pl.BlockSpec((D, NH * HD), lambda i: (0, 0)),
            pl.BlockSpec((1, D), lambda i: (0, 0)),
            pl.BlockSpec((E, D), lambda i: (0, 0)),
        ],
        out_specs=[
            pl.BlockSpec((bt, D), lambda i: (i, 0)),
            pl.BlockSpec((bt, D), lambda i: (i, 0)),
            pl.BlockSpec((bt, E), lambda i: (i, 0)),
        ],
        out_shape=[
            jax.ShapeDtypeStruct((T, D), _F32),
            jax.ShapeDtypeStruct((T, D), _F32),
            jax.ShapeDtypeStruct((T, E), _F32),
        ],
    )(attn, res, o_w, ln2_w.reshape(1, D), router_w)


def _moe_body(x_ref, w1_ref, w3_ref, w2_ref, comb_ref, out_ref):
    e = pl.program_id(1)
    x = x_ref[...]
    g = _dot(x, w1_ref[0])
    u = _dot(x, w3_ref[0])
    act = g * jax.nn.sigmoid(g) * u
    y = _dot(act, w2_ref[0])
    comb = comb_ref[...]
    lane = lax.broadcasted_iota(jnp.int32, comb.shape, 1)
    w = jnp.sum(jnp.where(lane == e, comb, 0.0), axis=1, keepdims=True)

    @pl.when(e == 0)
    def _init():
        out_ref[...] = jnp.zeros_like(out_ref)

    out_ref[...] += w * y


def _moe(h2, combine, w1, w3, w2, bt=256):
    grid = (T // bt, E)
    return pl.pallas_call(
        _moe_body,
        grid=grid,
        in_specs=[
            pl.BlockSpec((bt, D), lambda t, e: (t, 0)),
            pl.BlockSpec((1, D, F), lambda t, e: (e, 0, 0)),
            pl.BlockSpec((1, D, F), lambda t, e: (e, 0, 0)),
            pl.BlockSpec((1, F, D), lambda t, e: (e, 0, 0)),
            pl.BlockSpec((bt, E), lambda t, e: (t, 0)),
        ],
        out_specs=pl.BlockSpec((bt, D), lambda t, e: (t, 0)),
        out_shape=jax.ShapeDtypeStruct((T, D), _F32),
    )(h2, w1, w3, w2, combine)


def kernel(positions, hidden_states, kv_cache, residual, ln1_w, qkv_w, o_w,
           ln2_w, router_w, w1, w3, w2):
    res, qkv = _fused_qkv(hidden_states, residual, qkv_w, ln1_w)
    attn = _attention(qkv)
    res2, h2, combine = _post_attn(attn, res, o_w, ln2_w, router_w)
    ff = _moe(h2, combine, w1, w3, w2)
    return (ff, res2)


# speedup vs baseline: 1.3417x; 1.0353x over previous
import functools

import jax
import jax.numpy as jnp
from jax import lax
from jax.experimental import pallas as pl
from jax.experimental.pallas import tpu as pltpu
from jax.experimental.pallas import tpu_sc as plsc

T = 2048
D = 1024
NH = 16
NKV = 8
HD = 64
E = 8
TOPK = 2
F = 512
EPS = 1e-06
B = 256
NBLK = 24
P = NBLK * B
P2 = T * TOPK

_F32 = jnp.float32


_BF16 = jnp.bfloat16


def _dot_t(a, b):
    return lax.dot_general(a.astype(_BF16), b.astype(_BF16),
                           (((1,), (1,)), ((), ())),
                           preferred_element_type=_F32)


def _dot(a, b):
    return lax.dot_general(a.astype(_BF16), b.astype(_BF16),
                           (((1,), (0,)), ((), ())),
                           preferred_element_type=_F32)


def _qkv_body(h_ref, r_ref, w_ref, ln_ref, res_ref, qkv_ref):
    res = h_ref[...] + r_ref[...]
    res_ref[...] = res
    ms = jnp.mean(res * res, axis=1, keepdims=True)
    hn = res * lax.rsqrt(ms + EPS) * ln_ref[...]
    qkv_ref[...] = _dot_t(hn, w_ref[...])


def _fused_qkv(hidden, residual, qkv_w, ln1_w, bt=256):
    grid = (T // bt,)
    return pl.pallas_call(
        _qkv_body,
        grid=grid,
        in_specs=[
            pl.BlockSpec((bt, D), lambda i: (i, 0)),
            pl.BlockSpec((bt, D), lambda i: (i, 0)),
            pl.BlockSpec((NH * HD + 2 * NKV * HD, D), lambda i: (0, 0)),
            pl.BlockSpec((1, D), lambda i: (0, 0)),
        ],
        out_specs=[
            pl.BlockSpec((bt, D), lambda i: (i, 0)),
            pl.BlockSpec((bt, NH * HD + 2 * NKV * HD), lambda i: (i, 0)),
        ],
        out_shape=[
            jax.ShapeDtypeStruct((T, D), _F32),
            jax.ShapeDtypeStruct((T, NH * HD + 2 * NKV * HD), _F32),
        ],
    )(hidden, residual, qkv_w, ln1_w.reshape(1, D))


def _attn_body(qkv_ref, k2_ref, v2_ref, o_ref, s_scr, m_scr, l_scr, a_scr,
               *, qb, kb, nk):
    g = pl.program_id(0)
    qi = pl.program_id(1)
    scale = HD ** -0.5
    fmin = jnp.finfo(_F32).min

    def head(hh, kh):
        q = qkv_ref[:, pl.ds(64 * hh, 64)]
        m_scr[...] = jnp.full_like(m_scr, fmin)
        l_scr[...] = jnp.zeros_like(l_scr)
        for kc in range(nk):
            @pl.when(kc <= qi)
            def _s():
                k = k2_ref[pl.ds(kc * kb, kb), pl.ds(64 * kh, 64)]
                s = _dot_t(q, k) * scale
                rows = qi * qb + lax.broadcasted_iota(jnp.int32, s.shape, 0)
                cols = kc * kb + lax.broadcasted_iota(jnp.int32, s.shape, 1)
                s = jnp.where(cols <= rows, s, fmin)
                s_scr[:, pl.ds(kc * kb, kb)] = s
                m_scr[:, pl.ds(kc, 1)] = jnp.max(s, axis=1, keepdims=True)

        m = jnp.max(m_scr[...], axis=1, keepdims=True)
        for kc in range(nk):
            @pl.when(kc <= qi)
            def _p():
                p = jnp.exp(s_scr[:, pl.ds(kc * kb, kb)] - m)
                s_scr[:, pl.ds(kc * kb, kb)] = p
                l_scr[:, pl.ds(kc, 1)] = jnp.sum(p, axis=1, keepdims=True)

        l = jnp.sum(l_scr[...], axis=1, keepdims=True)
        a_scr[...] = jnp.zeros_like(a_scr)
        for kc in range(nk):
            @pl.when(kc <= qi)
            def _pv():
                probs = s_scr[:, pl.ds(kc * kb, kb)] / l
                v = v2_ref[pl.ds(kc * kb, kb), pl.ds(64 * kh, 64)]
                a_scr[...] += _dot(probs, v)
        o_ref[:, pl.ds(64 * hh, 64)] = a_scr[...]

    @pl.when(g % 2 == 0)
    def _even():
        head(0, 0)
        head(1, 0)

    @pl.when(g % 2 == 1)
    def _odd():
        head(0, 1)
        head(1, 1)


def _attention(qkv, qb=512, kb=512):
    nq = T // qb
    nk = T // kb
    grid = (NH // 2, nq)
    body = functools.partial(_attn_body, qb=qb, kb=kb, nk=nk)
    return pl.pallas_call(
        body,
        grid=grid,
        in_specs=[
            pl.BlockSpec((qb, 128), lambda g, qi: (qi, g)),
            pl.BlockSpec((T, 128), lambda g, qi: (0, 8 + g // 2)),
            pl.BlockSpec((T, 128), lambda g, qi: (0, 12 + g // 2)),
        ],
        out_specs=pl.BlockSpec((qb, 128), lambda g, qi: (qi, g)),
        out_shape=jax.ShapeDtypeStruct((T, NH * HD), _F32),
        scratch_shapes=[
            pltpu.VMEM((qb, T), _F32),
            pltpu.VMEM((qb, nk), _F32),
            pltpu.VMEM((qb, nk), _F32),
            pltpu.VMEM((qb, HD), _F32),
        ],
    )(qkv, qkv, qkv)


def _post_body(a_ref, res_ref, ow_ref, ln_ref, rw_ref,
               res2_ref, h2_ref, comb_ref):
    o = _dot_t(a_ref[...], ow_ref[...])
    res2 = o + res_ref[...]
    res2_ref[...] = res2
    ms = jnp.mean(res2 * res2, axis=1, keepdims=True)
    h2 = res2 * lax.rsqrt(ms + EPS) * ln_ref[...]
    h2_ref[...] = h2
    logits = _dot_t(h2, rw_ref[...])
    logits = logits - jnp.max(logits, axis=1, keepdims=True)
    ex = jnp.exp(logits)
    probs = ex / jnp.sum(ex, axis=1, keepdims=True)
    iota = lax.broadcasted_iota(jnp.int32, probs.shape, 1)
    m1 = jnp.max(probs, axis=1, keepdims=True)
    i1 = jnp.min(jnp.where(probs == m1, iota, E), axis=1, keepdims=True)
    oh1 = iota == i1
    probs2 = jnp.where(oh1, -1.0, probs)
    m2 = jnp.max(probs2, axis=1, keepdims=True)
    i2 = jnp.min(jnp.where(probs2 == m2, iota, E), axis=1, keepdims=True)
    oh2 = iota == i2
    comb_ref[...] = jnp.where(oh1 | oh2, probs, 0.0)


def _post_attn(attn, res, o_w, ln2_w, router_w, bt=256):
    grid = (T // bt,)
    return pl.pallas_call(
        _post_body,
        grid=grid,
        in_specs=[
            pl.BlockSpec((bt, NH * HD), lambda i: (i, 0)),
            pl.BlockSpec((bt, D), lambda i: (i, 0)),
            pl.BlockSpec((D, NH * HD), lambda i: (0, 0)),
            pl.BlockSpec((1, D), lambda i: (0, 0)),
            pl.BlockSpec((E, D), lambda i: (0, 0)),
        ],
        out_specs=[
            pl.BlockSpec((bt, D), lambda i: (i, 0)),
            pl.BlockSpec((bt, D), lambda i: (i, 0)),
            pl.BlockSpec((bt, E), lambda i: (i, 0)),
        ],
        out_shape=[
            jax.ShapeDtypeStruct((T, D), _F32),
            jax.ShapeDtypeStruct((T, D), _F32),
            jax.ShapeDtypeStruct((T, E), _F32),
        ],
    )(attn, res, o_w, ln2_w.reshape(1, D), router_w)


def _dispatch_body(comb_ref, rank_ref, topw_ref, be_ref):
    comb = comb_ref[...]
    iota = lax.broadcasted_iota(jnp.int32, comb.shape, 1)
    m1 = jnp.max(comb, axis=1, keepdims=True)
    i1 = jnp.min(jnp.where(comb == m1, iota, E), axis=1, keepdims=True)
    oh1 = iota == i1
    c2 = jnp.where(oh1, -1.0, comb)
    m2 = jnp.max(c2, axis=1, keepdims=True)
    i2 = jnp.min(jnp.where(c2 == m2, iota, E), axis=1, keepdims=True)
    oh2 = iota == i2
    s01 = jnp.where(oh1 | oh2, 1.0, 0.0)
    cb = 256
    li = lax.broadcasted_iota(jnp.int32, (cb, cb), 0)
    lj = lax.broadcasted_iota(jnp.int32, (cb, cb), 1)
    lstrict = jnp.where(lj < li, 1.0, 0.0)
    run = jnp.zeros((1, E), _F32)
    chunks = []
    for ci in range(T // cb):
        sc = s01[ci * cb:(ci + 1) * cb, :]
        chunks.append(_dot(lstrict, sc) + run)
        run = run + jnp.sum(sc, axis=0, keepdims=True)
    cnt = jnp.concatenate(chunks, axis=0)
    padded = jnp.floor((run + (B - 1)) * (1.0 / B)) * B
    ui = lax.broadcasted_iota(jnp.int32, (E, E), 0)
    uj = lax.broadcasted_iota(jnp.int32, (E, E), 1)
    off = _dot(padded, jnp.where(ui < uj, 1.0, 0.0))
    cume = off + padded
    pos = cnt + off
    r0 = jnp.sum(jnp.where(oh1, pos, 0.0), axis=1, keepdims=True)
    r1 = jnp.sum(jnp.where(oh2, pos, 0.0), axis=1, keepdims=True)
    rank_ref[...] = jnp.concatenate([r0, r1], axis=1).astype(jnp.int32)
    w0 = jnp.sum(jnp.where(oh1, comb, 0.0), axis=1, keepdims=True)
    w1v = jnp.sum(jnp.where(oh2, comb, 0.0), axis=1, keepdims=True)
    topw_ref[...] = jnp.concatenate([w0, w1v], axis=1)
    sb = (B * lax.broadcasted_iota(jnp.int32, (1, 32), 1)).astype(_F32)
    be = jnp.sum(jnp.where(sb >= cume.reshape(E, 1), 1.0, 0.0), axis=0,
                 keepdims=True)
    be_ref[...] = be.astype(jnp.int32)


def _dispatch(combine):
    return pl.pallas_call(
        _dispatch_body,
        grid=(1,),
        in_specs=[pl.BlockSpec((T, E), lambda i: (0, 0))],
        out_specs=[
            pl.BlockSpec((T, 2), lambda i: (0, 0)),
            pl.BlockSpec((T, 2), lambda i: (0, 0)),
            pl.BlockSpec((1, 32), lambda i: (0, 0)),
        ],
        out_shape=[
            jax.ShapeDtypeStruct((T, 2), jnp.int32),
            jax.ShapeDtypeStruct((T, 2), _F32),
            jax.ShapeDtypeStruct((1, 32), jnp.int32),
        ],
    )(combine)


def _sc_scatter_x(h2, rank0, rank1):
    mesh = plsc.VectorSubcoreMesh(core_axis_name="c", subcore_axis_name="s")
    per = T // 32

    @functools.partial(
        pl.kernel, mesh=mesh,
        out_type=jax.ShapeDtypeStruct((P, D), _F32),
        scratch_types=[pltpu.VMEM((per, D), _F32),
                       pltpu.VMEM((per,), jnp.int32),
                       pltpu.VMEM((per,), jnp.int32)],
    )
    def k(h2_hbm, r0_hbm, r1_hbm, xs_hbm, xv, i0_v, i1_v):
        wid = lax.axis_index("s") * 2 + lax.axis_index("c")
        base = wid * per
        pltpu.sync_copy(h2_hbm.at[pl.ds(base, per)], xv)
        pltpu.sync_copy(r0_hbm.at[pl.ds(base, per)], i0_v)
        pltpu.sync_copy(r1_hbm.at[pl.ds(base, per)], i1_v)
        pltpu.sync_copy(xv, xs_hbm.at[i0_v])
        pltpu.sync_copy(xv, xs_hbm.at[i1_v])

    return k(h2, rank0, rank1)


def _sc_gather_rows(table, idx, n_rows):
    mesh = plsc.VectorSubcoreMesh(core_axis_name="c", subcore_axis_name="s")
    per = n_rows // 32
    cs = 64
    nch = per // cs

    @functools.partial(
        pl.kernel, mesh=mesh,
        out_type=jax.ShapeDtypeStruct((n_rows, D), _F32),
        scratch_types=[pltpu.VMEM((cs,), jnp.int32),
                       pltpu.VMEM((cs, D), _F32),
                       pltpu.SemaphoreType.DMA],
    )
    def k(table_hbm, idx_hbm, out_hbm, idx_v, rows_v, sem):
        wid = lax.axis_index("s") * 2 + lax.axis_index("c")
        base = wid * per
        for c in range(nch):
            pltpu.sync_copy(idx_hbm.at[pl.ds(base + c * cs, cs)], idx_v)
            pltpu.async_copy(table_hbm.at[idx_v], rows_v, sem).wait()
            pltpu.sync_copy(rows_v, out_hbm.at[pl.ds(base + c * cs, cs)])

    return k(table, idx)


def _moe_body(be_ref, x_ref, w1_ref, w3_ref, w2_ref, out_ref):
    b = pl.program_id(0)

    @pl.when(be_ref[b] < E)
    def _():
        x = x_ref[...]
        g = _dot(x, w1_ref[0])
        u = _dot(x, w3_ref[0])
        act = g * jax.nn.sigmoid(g) * u
        out_ref[...] = _dot(act, w2_ref[0])


def _moe_sparse(x_sorted, be32, w1, w3, w2):
    def wmap(b, be):
        return (jnp.minimum(be[b], E - 1), 0, 0)

    grid_spec = pltpu.PrefetchScalarGridSpec(
        num_scalar_prefetch=1,
        grid=(NBLK,),
        in_specs=[
            pl.BlockSpec((B, D), lambda b, be: (b, 0)),
            pl.BlockSpec((1, D, F), wmap),
            pl.BlockSpec((1, D, F), wmap),
            pl.BlockSpec((1, F, D), wmap),
        ],
        out_specs=pl.BlockSpec((B, D), lambda b, be: (b, 0)),
    )
    return pl.pallas_call(
        _moe_body,
        grid_spec=grid_spec,
        out_shape=jax.ShapeDtypeStruct((P, D), _F32),
    )(be32, x_sorted, w1, w3, w2)


def _pair_body(yp_ref, tw_ref, out_ref):
    yp = yp_ref[...]
    tw = tw_ref[...]
    out_ref[...] = tw[:, 0:1] * yp[:, :D] + tw[:, 1:2] * yp[:, D:]


def _pair_sum(y_pairs2, topw2, bt=256):
    return pl.pallas_call(
        _pair_body,
        grid=(T // bt,),
        in_specs=[pl.BlockSpec((bt, 2 * D), lambda i: (i, 0)),
                  pl.BlockSpec((bt, 2), lambda i: (i, 0))],
        out_specs=pl.BlockSpec((bt, D), lambda i: (i, 0)),
        out_shape=jax.ShapeDtypeStruct((T, D), _F32),
    )(y_pairs2, topw2)


def kernel(positions, hidden_states, kv_cache, residual, ln1_w, qkv_w, o_w,
           ln2_w, router_w, w1, w3, w2):
    res, qkv = _fused_qkv(hidden_states, residual, qkv_w, ln1_w)
    attn = _attention(qkv)
    res2, h2, combine = _post_attn(attn, res, o_w, ln2_w, router_w)
    rank2, topw2, be = _dispatch(combine)
    x_sorted = _sc_scatter_x(h2, rank2[:, 0], rank2[:, 1])
    y_exp = _moe_sparse(x_sorted, be.reshape(32), w1, w3, w2)
    y_pairs = _sc_gather_rows(y_exp, rank2.reshape(P2), P2)
    ff = _pair_sum(y_pairs.reshape(T, 2 * D), topw2)
    return (ff, res2)
